# Initial kernel scaffold; baseline (speedup 1.0000x reference)
import jax
import jax.numpy as jnp
from jax.experimental import pallas as pl


def kernel(node_states, edges, kernel, kernel_attention):
    raise NotImplementedError("write your pallas kernel here")


# speedup vs baseline: 13.5100x; 13.5100x over previous
import functools

import jax
import jax.numpy as jnp
from jax import lax
from jax.experimental import pallas as pl
from jax.experimental.pallas import tpu as pltpu
from jax.experimental.pallas import tpu_sc as plsc

N_NODES = 10000
NPAD = 10240
E_TOTAL = 320000
D = 128
NW = 32
EPW = E_TOTAL // NW
CH = 80
NCH = EPW // CH
RPW = NPAD // NW


def _tc1_body(ns_ref, w_ref, a_ref, h_ref, st_ref):
    hb = jnp.dot(ns_ref[...], w_ref[...], preferred_element_type=jnp.float32)
    h_ref[...] = hb
    st_ref[...] = jnp.dot(hb, a_ref[...], preferred_element_type=jnp.float32)


def _tc1(ns_pad, w, a128):
    return pl.pallas_call(
        _tc1_body,
        grid=(NPAD // 1024,),
        in_specs=[
            pl.BlockSpec((1024, D), lambda i: (i, 0)),
            pl.BlockSpec((D, D), lambda i: (0, 0)),
            pl.BlockSpec((D, D), lambda i: (0, 0)),
        ],
        out_specs=[
            pl.BlockSpec((1024, D), lambda i: (i, 0)),
            pl.BlockSpec((1024, D), lambda i: (i, 0)),
        ],
        out_shape=[
            jax.ShapeDtypeStruct((NPAD, D), jnp.float32),
            jax.ShapeDtypeStruct((NPAD, D), jnp.float32),
        ],
    )(ns_pad, w, a128)


@functools.partial(
    pl.kernel,
    out_type=[
        jax.ShapeDtypeStruct((NPAD, D), jnp.float32),
        jax.ShapeDtypeStruct((NPAD, D), jnp.float32),
        jax.ShapeDtypeStruct((NPAD,), jnp.float32),
        jax.ShapeDtypeStruct((NPAD,), jnp.float32),
    ],
    mesh=plsc.VectorSubcoreMesh(core_axis_name="c", subcore_axis_name="s"),
    compiler_params=pltpu.CompilerParams(needs_layout_passes=False),
    scratch_types=[
        pltpu.VMEM((NCH, CH), jnp.int32),
        pltpu.VMEM((NCH, CH), jnp.int32),
        pltpu.VMEM((CH,), jnp.float32),
        pltpu.VMEM((CH,), jnp.float32),
        pltpu.VMEM((CH,), jnp.float32),
        pltpu.VMEM((CH, D), jnp.float32),
        pltpu.VMEM((640,), jnp.float32),
        pltpu.VMEM_SHARED((NPAD, D), jnp.float32),
        pltpu.VMEM_SHARED((NPAD,), jnp.float32),
    ],
)
def _sc_main(h_hbm, sd_hbm, ss_hbm, dst_hbm, src_hbm,
             out0_hbm, out1_hbm, den0_hbm, den1_hbm,
             dst_i, src_i, sd_g, ss_g, w_g, rows, zbuf1d,
             out_acc, den_acc):
    c = lax.axis_index("c")
    s = lax.axis_index("s")
    wid = c * 16 + s

    pltpu.sync_copy(dst_hbm.at[wid], dst_i)
    pltpu.sync_copy(src_hbm.at[wid], src_i)

    zv = jnp.zeros((16,), jnp.float32)

    def zrow(r, carry):
        for f in range(8):
            rows[r, pl.ds(16 * f, 16)] = zv
        return carry

    lax.fori_loop(0, CH, zrow, 0)

    def z1(i, carry):
        zbuf1d[pl.ds(16 * i, 16)] = zv
        return carry

    lax.fori_loop(0, 40, z1, 0)

    def zacc(k, carry):
        pltpu.sync_copy(rows, out_acc.at[pl.ds(640 * s + 80 * k, 80), :])
        return carry

    lax.fori_loop(0, 8, zacc, 0)
    pltpu.sync_copy(zbuf1d, den_acc.at[pl.ds(640 * s, 640)])

    plsc.subcore_barrier()

    neg2 = jnp.full((16,), -2.0, jnp.float32)
    pos2 = jnp.full((16,), 2.0, jnp.float32)
    zero = jnp.zeros((16,), jnp.float32)
    slope = jnp.full((16,), 0.2, jnp.float32)

    def chunk(j, carry):
        pltpu.sync_copy(sd_hbm.at[dst_i.at[j]], sd_g)
        pltpu.sync_copy(ss_hbm.at[src_i.at[j]], ss_g)
        pltpu.sync_copy(h_hbm.at[src_i.at[j]], rows)

        for g in range(CH // 16):
            x = sd_g[pl.ds(16 * g, 16)] + ss_g[pl.ds(16 * g, 16)]
            x = jnp.where(x >= zero, x, slope * x)
            x = jnp.minimum(jnp.maximum(x, neg2), pos2)
            w_g[pl.ds(16 * g, 16)] = jnp.exp(x)

        def scale(e, c2):
            ev = jnp.full((16,), e, jnp.int32)
            wsp = plsc.load_gather(w_g, [ev])
            for f in range(8):
                rows[e, pl.ds(16 * f, 16)] = rows[e, pl.ds(16 * f, 16)] * wsp
            return c2

        lax.fori_loop(0, CH, scale, 0)
        pltpu.sync_copy(rows, out_acc.at[dst_i.at[j]], add=True)
        pltpu.sync_copy(w_g, den_acc.at[dst_i.at[j]], add=True)
        return carry

    lax.fori_loop(0, NCH, chunk, 0)

    plsc.subcore_barrier()

    @pl.when(c == 0)
    def _():
        pltpu.sync_copy(out_acc.at[pl.ds(640 * s, 640), :],
                        out0_hbm.at[pl.ds(640 * s, 640), :])
        pltpu.sync_copy(den_acc.at[pl.ds(640 * s, 640)],
                        den0_hbm.at[pl.ds(640 * s, 640)])

    @pl.when(c == 1)
    def _():
        pltpu.sync_copy(out_acc.at[pl.ds(640 * s, 640), :],
                        out1_hbm.at[pl.ds(640 * s, 640), :])
        pltpu.sync_copy(den_acc.at[pl.ds(640 * s, 640)],
                        den1_hbm.at[pl.ds(640 * s, 640)])


@functools.partial(
    pl.kernel,
    out_type=jax.ShapeDtypeStruct((NPAD, D), jnp.float32),
    mesh=plsc.VectorSubcoreMesh(core_axis_name="c", subcore_axis_name="s"),
    compiler_params=pltpu.CompilerParams(needs_layout_passes=False),
    scratch_types=[
        pltpu.VMEM((RPW,), jnp.float32),
        pltpu.VMEM((RPW,), jnp.float32),
        pltpu.VMEM((RPW,), jnp.float32),
        pltpu.VMEM((CH, D), jnp.float32),
        pltpu.VMEM((CH, D), jnp.float32),
    ],
)
def _sc_div(p0_hbm, p1_hbm, den0_hbm, den1_hbm, out_hbm, d0v, d1v, iv, pa, pb):
    c = lax.axis_index("c")
    s = lax.axis_index("s")
    wid = c * 16 + s
    r0 = wid * RPW
    pltpu.sync_copy(den0_hbm.at[pl.ds(r0, RPW)], d0v)
    pltpu.sync_copy(den1_hbm.at[pl.ds(r0, RPW)], d1v)

    eps = jnp.full((16,), 1e-7, jnp.float32)
    one = jnp.full((16,), 1.0, jnp.float32)

    def inv_g(i, carry):
        dv = d0v[pl.ds(16 * i, 16)] + d1v[pl.ds(16 * i, 16)] + eps
        iv[pl.ds(16 * i, 16)] = one / dv
        return carry

    lax.fori_loop(0, RPW // 16, inv_g, 0)

    def blk(k, carry):
        rb = r0 + CH * k
        pltpu.sync_copy(p0_hbm.at[pl.ds(rb, CH), :], pa)
        pltpu.sync_copy(p1_hbm.at[pl.ds(rb, CH), :], pb)

        def row(e, c2):
            ev = jnp.full((16,), CH * k + e, jnp.int32)
            isp = plsc.load_gather(iv, [ev])
            for f in range(8):
                pa[e, pl.ds(16 * f, 16)] = (
                    pa[e, pl.ds(16 * f, 16)] + pb[e, pl.ds(16 * f, 16)]
                ) * isp
            return c2

        lax.fori_loop(0, CH, row, 0)
        pltpu.sync_copy(pa, out_hbm.at[pl.ds(rb, CH), :])
        return carry

    lax.fori_loop(0, RPW // CH, blk, 0)


def kernel(node_states, edges, kernel, kernel_attention):
    ka = kernel_attention.reshape(2 * D)
    a128 = jnp.zeros((D, D), jnp.float32).at[:, 0].set(ka[:D]).at[:, 1].set(ka[D:])
    ns_pad = jnp.pad(node_states, ((0, NPAD - N_NODES), (0, 0)))
    h, st = _tc1(ns_pad, kernel, a128)
    sd = st[:, 0]
    ss = st[:, 1]
    e32 = edges.astype(jnp.int32)
    dst3 = e32[:, 0].reshape(NW, NCH, CH)
    src3 = e32[:, 1].reshape(NW, NCH, CH)
    p0, p1, den0, den1 = _sc_main(h, sd, ss, dst3, src3)
    outp = _sc_div(p0, p1, den0, den1)
    return outp[:N_NODES]


# speedup vs baseline: 15.9308x; 1.1792x over previous
import functools

import jax
import jax.numpy as jnp
from jax import lax
from jax.experimental import pallas as pl
from jax.experimental.pallas import tpu as pltpu
from jax.experimental.pallas import tpu_sc as plsc

N_NODES = 10000
NPAD = 10240
E_TOTAL = 320000
D = 128
NW = 32
EPW = E_TOTAL // NW
CH = 80
NCH = EPW // CH
RPW = NPAD // NW


def _tc1_body(ns_ref, w_ref, a_ref, h_ref, st_ref):
    hb = jnp.dot(ns_ref[...], w_ref[...], preferred_element_type=jnp.float32)
    h_ref[...] = hb
    st_ref[...] = jnp.dot(hb, a_ref[...], preferred_element_type=jnp.float32)


def _tc1(ns_pad, w, a128):
    return pl.pallas_call(
        _tc1_body,
        grid=(NPAD // 1024,),
        in_specs=[
            pl.BlockSpec((1024, D), lambda i: (i, 0)),
            pl.BlockSpec((D, D), lambda i: (0, 0)),
            pl.BlockSpec((D, D), lambda i: (0, 0)),
        ],
        out_specs=[
            pl.BlockSpec((1024, D), lambda i: (i, 0)),
            pl.BlockSpec((1024, D), lambda i: (i, 0)),
        ],
        out_shape=[
            jax.ShapeDtypeStruct((NPAD, D), jnp.float32),
            jax.ShapeDtypeStruct((NPAD, D), jnp.float32),
        ],
    )(ns_pad, w, a128)


@functools.partial(
    pl.kernel,
    out_type=[
        jax.ShapeDtypeStruct((NPAD, D), jnp.float32),
        jax.ShapeDtypeStruct((NPAD, D), jnp.float32),
        jax.ShapeDtypeStruct((NPAD,), jnp.float32),
        jax.ShapeDtypeStruct((NPAD,), jnp.float32),
    ],
    mesh=plsc.VectorSubcoreMesh(core_axis_name="c", subcore_axis_name="s"),
    compiler_params=pltpu.CompilerParams(needs_layout_passes=False),
    scratch_types=[
        pltpu.VMEM((1, CH), jnp.int32),
        pltpu.VMEM((1, CH), jnp.int32),
        pltpu.VMEM((1, CH), jnp.int32),
        pltpu.VMEM((1, CH), jnp.int32),
        pltpu.VMEM((CH,), jnp.float32),
        pltpu.VMEM((CH,), jnp.float32),
        pltpu.VMEM((CH,), jnp.float32),
        pltpu.VMEM((CH,), jnp.float32),
        pltpu.VMEM((CH,), jnp.float32),
        pltpu.VMEM((CH,), jnp.float32),
        pltpu.VMEM((CH, D), jnp.float32),
        pltpu.VMEM((CH, D), jnp.float32),
        pltpu.SemaphoreType.DMA,
        pltpu.SemaphoreType.DMA,
        pltpu.SemaphoreType.DMA,
        pltpu.SemaphoreType.DMA,
        pltpu.VMEM_SHARED((NPAD, D), jnp.float32),
        pltpu.VMEM_SHARED((NPAD,), jnp.float32),
    ],
)
def _sc_main(h_hbm, sd_hbm, ss_hbm, dst_hbm, src_hbm,
             out0_hbm, out1_hbm, den0_hbm, den1_hbm,
             di_a, di_b, si_a, si_b, sd_a, sd_b, ss_a, ss_b, w_a, w_b,
             rows_a, rows_b, gsem_a, gsem_b, ssem_a, ssem_b,
             out_acc, den_acc):
    c = lax.axis_index("c")
    s = lax.axis_index("s")
    wid = c * 16 + s

    zv = jnp.zeros((16,), jnp.float32)

    def zrow(r, carry):
        for f in range(8):
            rows_a[r, pl.ds(16 * f, 16)] = zv
        return carry

    lax.fori_loop(0, CH, zrow, 0)
    for g in range(CH // 16):
        w_a[pl.ds(16 * g, 16)] = zv

    def zacc(k, carry):
        pltpu.sync_copy(rows_a, out_acc.at[pl.ds(640 * s + 80 * k, 80), :])
        pltpu.sync_copy(w_a, den_acc.at[pl.ds(640 * s + 80 * k, 80)])
        return carry

    lax.fori_loop(0, 8, zacc, 0)

    plsc.subcore_barrier()

    neg2 = jnp.full((16,), -2.0, jnp.float32)
    pos2 = jnp.full((16,), 2.0, jnp.float32)
    zero = jnp.zeros((16,), jnp.float32)
    slope = jnp.full((16,), 0.2, jnp.float32)

    def start_g(j, dib, sib, sdb, ssb, rb, sem):
        pltpu.sync_copy(dst_hbm.at[wid, j], dib)
        pltpu.sync_copy(src_hbm.at[wid, j], sib)
        pltpu.async_copy(sd_hbm.at[dib.at[0]], sdb, sem)
        pltpu.async_copy(ss_hbm.at[sib.at[0]], ssb, sem)
        pltpu.async_copy(h_hbm.at[sib.at[0]], rb, sem)

    def wait_g(dib, sib, sdb, ssb, rb, sem):
        pltpu.make_async_copy(sd_hbm.at[dib.at[0]], sdb, sem).wait()
        pltpu.make_async_copy(ss_hbm.at[sib.at[0]], ssb, sem).wait()
        pltpu.make_async_copy(h_hbm.at[sib.at[0]], rb, sem).wait()

    def compute(sdb, ssb, wb, rb):
        for g in range(CH // 16):
            x = sdb[pl.ds(16 * g, 16)] + ssb[pl.ds(16 * g, 16)]
            x = jnp.where(x >= zero, x, slope * x)
            x = jnp.minimum(jnp.maximum(x, neg2), pos2)
            wb[pl.ds(16 * g, 16)] = jnp.exp(x)

        def scale(e, c2):
            ev = jnp.full((16,), e, jnp.int32)
            wsp = plsc.load_gather(wb, [ev])
            for f in range(8):
                rb[e, pl.ds(16 * f, 16)] = rb[e, pl.ds(16 * f, 16)] * wsp
            return c2

        lax.fori_loop(0, CH, scale, 0)

    def start_s(dib, wb, rb, sem):
        pltpu.async_copy(rb, out_acc.at[dib.at[0]], sem, add=True)
        pltpu.async_copy(wb, den_acc.at[dib.at[0]], sem, add=True)

    def wait_s(dib, wb, rb, sem):
        pltpu.make_async_copy(rb, out_acc.at[dib.at[0]], sem).wait()
        pltpu.make_async_copy(wb, den_acc.at[dib.at[0]], sem).wait()

    bufs = ((di_a, si_a, sd_a, ss_a, w_a, rows_a, gsem_a, ssem_a),
            (di_b, si_b, sd_b, ss_b, w_b, rows_b, gsem_b, ssem_b))

    def proc(j, p, first, last):
        dib, sib, sdb, ssb, wb, rb, gsem, ssem = bufs[p]
        diq, siq, sdq, ssq, wq, rq, gsemq, ssemq = bufs[1 - p]
        wait_g(dib, sib, sdb, ssb, rb, gsem)
        compute(sdb, ssb, wb, rb)
        start_s(dib, wb, rb, ssem)
        if not last:
            if not first:
                wait_s(diq, wq, rq, ssemq)
            start_g(j + 1, diq, siq, sdq, ssq, rq, gsemq)

    start_g(0, di_a, si_a, sd_a, ss_a, rows_a, gsem_a)
    proc(0, 0, True, False)

    def pair(jj, carry):
        proc(2 * jj + 1, 1, False, False)
        proc(2 * jj + 2, 0, False, False)
        return carry

    lax.fori_loop(0, 61, pair, 0)
    proc(123, 1, False, False)
    proc(124, 0, False, True)

    wait_s(di_b, w_b, rows_b, ssem_b)
    wait_s(di_a, w_a, rows_a, ssem_a)

    plsc.subcore_barrier()

    @pl.when(c == 0)
    def _():
        pltpu.sync_copy(out_acc.at[pl.ds(640 * s, 640), :],
                        out0_hbm.at[pl.ds(640 * s, 640), :])
        pltpu.sync_copy(den_acc.at[pl.ds(640 * s, 640)],
                        den0_hbm.at[pl.ds(640 * s, 640)])

    @pl.when(c == 1)
    def _():
        pltpu.sync_copy(out_acc.at[pl.ds(640 * s, 640), :],
                        out1_hbm.at[pl.ds(640 * s, 640), :])
        pltpu.sync_copy(den_acc.at[pl.ds(640 * s, 640)],
                        den1_hbm.at[pl.ds(640 * s, 640)])


@functools.partial(
    pl.kernel,
    out_type=jax.ShapeDtypeStruct((NPAD, D), jnp.float32),
    mesh=plsc.VectorSubcoreMesh(core_axis_name="c", subcore_axis_name="s"),
    compiler_params=pltpu.CompilerParams(needs_layout_passes=False),
    scratch_types=[
        pltpu.VMEM((RPW,), jnp.float32),
        pltpu.VMEM((RPW,), jnp.float32),
        pltpu.VMEM((RPW,), jnp.float32),
        pltpu.VMEM((CH, D), jnp.float32),
        pltpu.VMEM((CH, D), jnp.float32),
    ],
)
def _sc_div(p0_hbm, p1_hbm, den0_hbm, den1_hbm, out_hbm, d0v, d1v, iv, pa, pb):
    c = lax.axis_index("c")
    s = lax.axis_index("s")
    wid = c * 16 + s
    r0 = wid * RPW
    pltpu.sync_copy(den0_hbm.at[pl.ds(r0, RPW)], d0v)
    pltpu.sync_copy(den1_hbm.at[pl.ds(r0, RPW)], d1v)

    eps = jnp.full((16,), 1e-7, jnp.float32)
    one = jnp.full((16,), 1.0, jnp.float32)

    def inv_g(i, carry):
        dv = d0v[pl.ds(16 * i, 16)] + d1v[pl.ds(16 * i, 16)] + eps
        iv[pl.ds(16 * i, 16)] = one / dv
        return carry

    lax.fori_loop(0, RPW // 16, inv_g, 0)

    def blk(k, carry):
        rb = r0 + CH * k
        pltpu.sync_copy(p0_hbm.at[pl.ds(rb, CH), :], pa)
        pltpu.sync_copy(p1_hbm.at[pl.ds(rb, CH), :], pb)

        def row(e, c2):
            ev = jnp.full((16,), CH * k + e, jnp.int32)
            isp = plsc.load_gather(iv, [ev])
            for f in range(8):
                pa[e, pl.ds(16 * f, 16)] = (
                    pa[e, pl.ds(16 * f, 16)] + pb[e, pl.ds(16 * f, 16)]
                ) * isp
            return c2

        lax.fori_loop(0, CH, row, 0)
        pltpu.sync_copy(pa, out_hbm.at[pl.ds(rb, CH), :])
        return carry

    lax.fori_loop(0, RPW // CH, blk, 0)


def kernel(node_states, edges, kernel, kernel_attention):
    ka = kernel_attention.reshape(2 * D)
    a128 = jnp.zeros((D, D), jnp.float32).at[:, 0].set(ka[:D]).at[:, 1].set(ka[D:])
    ns_pad = jnp.pad(node_states, ((0, NPAD - N_NODES), (0, 0)))
    h, st = _tc1(ns_pad, kernel, a128)
    sd = st[:, 0]
    ss = st[:, 1]
    e32 = edges.astype(jnp.int32)
    dst3 = e32[:, 0].reshape(NW, NCH, 1, CH)
    src3 = e32[:, 1].reshape(NW, NCH, 1, CH)
    p0, p1, den0, den1 = _sc_main(h, sd, ss, dst3, src3)
    outp = _sc_div(p0, p1, den0, den1)
    return outp[:N_NODES]


# speedup vs baseline: 21.7590x; 1.3658x over previous
import functools

import jax
import jax.numpy as jnp
from jax import lax
from jax.experimental import pallas as pl
from jax.experimental.pallas import tpu as pltpu
from jax.experimental.pallas import tpu_sc as plsc

N_NODES = 10000
NPAD = 10240
E_TOTAL = 320000
D = 128
NW = 32
EPW = E_TOTAL // NW
CH = 80
NCH = EPW // CH
RPW = NPAD // NW


def _tc1_body(ns_ref, w_ref, a_ref, h_ref, st_ref):
    hb = jnp.dot(ns_ref[...], w_ref[...], preferred_element_type=jnp.float32)
    h_ref[...] = hb
    st_ref[...] = jnp.dot(hb, a_ref[...], preferred_element_type=jnp.float32)


def _tc1(ns, w, a128):
    return pl.pallas_call(
        _tc1_body,
        grid=(10,),
        in_specs=[
            pl.BlockSpec((1000, D), lambda i: (i, 0)),
            pl.BlockSpec((D, D), lambda i: (0, 0)),
            pl.BlockSpec((D, D), lambda i: (0, 0)),
        ],
        out_specs=[
            pl.BlockSpec((1000, D), lambda i: (i, 0)),
            pl.BlockSpec((1000, D), lambda i: (i, 0)),
        ],
        out_shape=[
            jax.ShapeDtypeStruct((N_NODES, D), jnp.float32),
            jax.ShapeDtypeStruct((N_NODES, D), jnp.float32),
        ],
    )(ns, w, a128)


@functools.partial(
    pl.kernel,
    out_type=[
        jax.ShapeDtypeStruct((NPAD, D), jnp.float32),
        jax.ShapeDtypeStruct((NPAD, D), jnp.float32),
        jax.ShapeDtypeStruct((NPAD,), jnp.float32),
        jax.ShapeDtypeStruct((NPAD,), jnp.float32),
    ],
    mesh=plsc.VectorSubcoreMesh(core_axis_name="c", subcore_axis_name="s"),
    compiler_params=pltpu.CompilerParams(needs_layout_passes=False),
    scratch_types=[
        [pltpu.VMEM((1, CH), jnp.int32)] * 4,
        [pltpu.VMEM((1, CH), jnp.int32)] * 4,
        [pltpu.SemaphoreType.DMA] * 4,
        [pltpu.VMEM((CH,), jnp.float32)] * 2,
        [pltpu.VMEM((CH,), jnp.float32)] * 2,
        [pltpu.VMEM((CH,), jnp.float32)] * 2,
        [pltpu.VMEM((CH, D), jnp.float32)] * 2,
        [pltpu.SemaphoreType.DMA] * 2,
        [pltpu.SemaphoreType.DMA] * 2,
        pltpu.VMEM_SHARED((NPAD, D), jnp.float32),
        pltpu.VMEM_SHARED((NPAD,), jnp.float32),
    ],
)
def _sc_main(h_hbm, sd_hbm, ss_hbm, dst_hbm, src_hbm,
             out0_hbm, out1_hbm, den0_hbm, den1_hbm,
             di, si, isem, sd2, ss2, w2, rows2, gsem, ssem,
             out_acc, den_acc):
    c = lax.axis_index("c")
    s = lax.axis_index("s")
    wid = c * 16 + s

    zv = jnp.zeros((16,), jnp.float32)
    rows_a = rows2[0]
    w_a = w2[0]

    def zrow(r, carry):
        for f in range(8):
            rows_a[r, pl.ds(16 * f, 16)] = zv
        return carry

    lax.fori_loop(0, CH, zrow, 0)
    for g in range(CH // 16):
        w_a[pl.ds(16 * g, 16)] = zv

    def zacc(k, carry):
        pltpu.sync_copy(rows_a, out_acc.at[pl.ds(640 * s + 80 * k, 80), :])
        pltpu.sync_copy(w_a, den_acc.at[pl.ds(640 * s + 80 * k, 80)])
        return carry

    lax.fori_loop(0, 8, zacc, 0)

    plsc.subcore_barrier()

    neg2 = jnp.full((16,), -2.0, jnp.float32)
    pos2 = jnp.full((16,), 2.0, jnp.float32)
    zero = jnp.zeros((16,), jnp.float32)
    slope = jnp.full((16,), 0.2, jnp.float32)

    def idx_start(j, sl):
        pltpu.async_copy(dst_hbm.at[wid, j], di[sl], isem[sl])
        pltpu.async_copy(src_hbm.at[wid, j], si[sl], isem[sl])

    def idx_wait(j, sl):
        pltpu.make_async_copy(dst_hbm.at[wid, j], di[sl], isem[sl]).wait()
        pltpu.make_async_copy(src_hbm.at[wid, j], si[sl], isem[sl]).wait()

    def start_g(j, sl, p):
        idx_wait(j, sl)
        pltpu.async_copy(sd_hbm.at[di[sl].at[0]], sd2[p], gsem[p])
        pltpu.async_copy(ss_hbm.at[si[sl].at[0]], ss2[p], gsem[p])
        pltpu.async_copy(h_hbm.at[si[sl].at[0]], rows2[p], gsem[p])

    def wait_g(sl, p):
        pltpu.make_async_copy(sd_hbm.at[di[sl].at[0]], sd2[p], gsem[p]).wait()
        pltpu.make_async_copy(ss_hbm.at[si[sl].at[0]], ss2[p], gsem[p]).wait()
        pltpu.make_async_copy(h_hbm.at[si[sl].at[0]], rows2[p], gsem[p]).wait()

    def compute(p):
        sdb, ssb, wb, rb = sd2[p], ss2[p], w2[p], rows2[p]
        for g in range(CH // 16):
            x = sdb[pl.ds(16 * g, 16)] + ssb[pl.ds(16 * g, 16)]
            x = jnp.where(x >= zero, x, slope * x)
            x = jnp.minimum(jnp.maximum(x, neg2), pos2)
            wb[pl.ds(16 * g, 16)] = jnp.exp(x)

        def scale(e4, c2):
            for u in range(4):
                e = 4 * e4 + u
                ev = jnp.full((16,), e, jnp.int32)
                wsp = plsc.load_gather(wb, [ev])
                for f in range(8):
                    rb[e, pl.ds(16 * f, 16)] = rb[e, pl.ds(16 * f, 16)] * wsp
            return c2

        lax.fori_loop(0, CH // 4, scale, 0)

    def start_s(sl, p):
        pltpu.async_copy(rows2[p], out_acc.at[di[sl].at[0]], ssem[p], add=True)
        pltpu.async_copy(w2[p], den_acc.at[di[sl].at[0]], ssem[p], add=True)

    def wait_s(sl, p):
        pltpu.make_async_copy(rows2[p], out_acc.at[di[sl].at[0]], ssem[p]).wait()
        pltpu.make_async_copy(w2[p], den_acc.at[di[sl].at[0]], ssem[p]).wait()

    def proc(j, sl, first=False, last=False):
        p = sl % 2
        if not last:
            pl.when(j + 2 < NCH)(lambda: idx_start(j + 2, (sl + 2) % 4))
        wait_g(sl, p)
        compute(p)
        start_s(sl, p)
        if not last:
            if not first:
                pl.when(j - 1 >= 0)(lambda: wait_s((sl + 3) % 4, 1 - p))
            pl.when(j + 1 < NCH)(lambda: start_g(j + 1, (sl + 1) % 4, 1 - p))

    idx_start(0, 0)
    idx_start(1, 1)
    start_g(0, 0, 0)
    proc(0, 0, first=True)

    def quad(jj, carry):
        j = 4 * jj + 1
        proc(j, 1)
        proc(j + 1, 2)
        proc(j + 2, 3)
        proc(j + 3, 0)
        return carry

    lax.fori_loop(0, 31, quad, 0)

    wait_s(0, 0)

    plsc.subcore_barrier()

    @pl.when(c == 0)
    def _():
        pltpu.sync_copy(out_acc.at[pl.ds(640 * s, 640), :],
                        out0_hbm.at[pl.ds(640 * s, 640), :])
        pltpu.sync_copy(den_acc.at[pl.ds(640 * s, 640)],
                        den0_hbm.at[pl.ds(640 * s, 640)])

    @pl.when(c == 1)
    def _():
        pltpu.sync_copy(out_acc.at[pl.ds(640 * s, 640), :],
                        out1_hbm.at[pl.ds(640 * s, 640), :])
        pltpu.sync_copy(den_acc.at[pl.ds(640 * s, 640)],
                        den1_hbm.at[pl.ds(640 * s, 640)])


@functools.partial(
    pl.kernel,
    out_type=jax.ShapeDtypeStruct((N_NODES, D), jnp.float32),
    mesh=plsc.VectorSubcoreMesh(core_axis_name="c", subcore_axis_name="s"),
    compiler_params=pltpu.CompilerParams(needs_layout_passes=False),
    scratch_types=[
        pltpu.VMEM((RPW,), jnp.float32),
        pltpu.VMEM((RPW,), jnp.float32),
        pltpu.VMEM((RPW,), jnp.float32),
        pltpu.VMEM((CH, D), jnp.float32),
        pltpu.VMEM((CH, D), jnp.float32),
    ],
)
def _sc_div(p0_hbm, p1_hbm, den0_hbm, den1_hbm, out_hbm, d0v, d1v, iv, pa, pb):
    c = lax.axis_index("c")
    s = lax.axis_index("s")
    wid = c * 16 + s
    r0 = wid * RPW
    pltpu.sync_copy(den0_hbm.at[pl.ds(r0, RPW)], d0v)
    pltpu.sync_copy(den1_hbm.at[pl.ds(r0, RPW)], d1v)

    eps = jnp.full((16,), 1e-7, jnp.float32)
    one = jnp.full((16,), 1.0, jnp.float32)

    def inv_g(i, carry):
        dv = d0v[pl.ds(16 * i, 16)] + d1v[pl.ds(16 * i, 16)] + eps
        iv[pl.ds(16 * i, 16)] = one / dv
        return carry

    lax.fori_loop(0, RPW // 16, inv_g, 0)

    nblk = jnp.where(wid == NW - 1, 1, RPW // CH)

    def blk(k, carry):
        rb = r0 + CH * k
        pltpu.sync_copy(p0_hbm.at[pl.ds(rb, CH), :], pa)
        pltpu.sync_copy(p1_hbm.at[pl.ds(rb, CH), :], pb)

        def row(e, c2):
            ev = jnp.full((16,), CH * k + e, jnp.int32)
            isp = plsc.load_gather(iv, [ev])
            for f in range(8):
                pa[e, pl.ds(16 * f, 16)] = (
                    pa[e, pl.ds(16 * f, 16)] + pb[e, pl.ds(16 * f, 16)]
                ) * isp
            return c2

        lax.fori_loop(0, CH, row, 0)
        pltpu.sync_copy(pa, out_hbm.at[pl.ds(rb, CH), :])
        return carry

    lax.fori_loop(0, nblk, blk, 0)


def kernel(node_states, edges, kernel, kernel_attention):
    ka = kernel_attention.reshape(2 * D)
    a128 = jnp.zeros((D, D), jnp.float32).at[:, 0].set(ka[:D]).at[:, 1].set(ka[D:])
    h, st = _tc1(node_states, kernel, a128)
    sd = st[:, 0]
    ss = st[:, 1]
    e32 = edges.astype(jnp.int32)
    dst3 = e32[:, 0].reshape(NW, NCH, 1, CH)
    src3 = e32[:, 1].reshape(NW, NCH, 1, CH)
    p0, p1, den0, den1 = _sc_main(h, sd, ss, dst3, src3)
    return _sc_div(p0, p1, den0, den1)
